# Initial kernel scaffold; baseline (speedup 1.0000x reference)
import functools

import jax
import jax.numpy as jnp
from jax import lax
from jax.experimental import pallas as pl
from jax.experimental.pallas import tpu as pltpu
from jax.experimental.pallas import tpu_sc as plsc

BATCH = 4096
SEQ = 200
D_EMB = 32
B = BATCH * SEQ

NC = 2
NS = 16
NW = NC * NS

IDX_ROW = 128
ROWS_PER_CHUNK = 8
CHUNK = IDX_ROW * ROWS_PER_CHUNK
B_PER_W = B // NW
N_CHUNKS = B_PER_W // CHUNK
IDX_ROWS_PER_W = B_PER_W // IDX_ROW


def _emb_body(idx_hbm, table_hbm, out_hbm, idx_v, rows_v, gsem):
    wid = lax.axis_index("s") * NC + lax.axis_index("c")
    idx_row_base = wid * IDX_ROWS_PER_W
    out_base = wid * B_PER_W

    def chunk_body(c, _):
        pltpu.sync_copy(
            idx_hbm.at[pl.ds(idx_row_base + c * ROWS_PER_CHUNK, ROWS_PER_CHUNK)],
            idx_v,
        )
        descs = []
        for r in range(ROWS_PER_CHUNK):
            descs.append(
                pltpu.async_copy(
                    table_hbm.at[idx_v.at[r]],
                    rows_v.at[pl.ds(r * IDX_ROW, IDX_ROW)],
                    gsem,
                )
            )
        for d in descs:
            d.wait()
        pltpu.sync_copy(rows_v, out_hbm.at[pl.ds(out_base + c * CHUNK, CHUNK)])
        return 0

    lax.fori_loop(0, N_CHUNKS, chunk_body, 0)


def _emb_lookup(idx2d, table):
    mesh = plsc.VectorSubcoreMesh(core_axis_name="c", subcore_axis_name="s")
    k = functools.partial(
        pl.kernel,
        mesh=mesh,
        out_type=jax.ShapeDtypeStruct((B, D_EMB), jnp.float32),
        scratch_types=[
            pltpu.VMEM((ROWS_PER_CHUNK, IDX_ROW), jnp.int32),
            pltpu.VMEM((CHUNK, D_EMB), jnp.float32),
            pltpu.SemaphoreType.DMA,
        ],
    )(_emb_body)
    return k(idx2d, table)


def kernel(batch_words, emb_weight):
    idx2d = batch_words.reshape(B // IDX_ROW, IDX_ROW).astype(jnp.int32)
    out = _emb_lookup(idx2d, emb_weight)
    return out.reshape(BATCH, SEQ, D_EMB)


# speedup vs baseline: 1.4572x; 1.4572x over previous
import functools

import jax
import jax.numpy as jnp
from jax import lax
from jax.experimental import pallas as pl
from jax.experimental.pallas import tpu as pltpu
from jax.experimental.pallas import tpu_sc as plsc

BATCH = 4096
SEQ = 200
D_EMB = 32
B = BATCH * SEQ

NC = 2
NS = 16
NW = NC * NS

IDX_ROW = 128
ROWS_PER_CHUNK = 8
CHUNK = IDX_ROW * ROWS_PER_CHUNK
B_PER_W = B // NW
N_CHUNKS = B_PER_W // CHUNK
IDX_ROWS_PER_W = B_PER_W // IDX_ROW


def _emb_body(idx_hbm, table_hbm, out_hbm, idx_v, rows_v, gsem):
    wid = lax.axis_index("s") * NC + lax.axis_index("c")
    idx_row_base = wid * IDX_ROWS_PER_W
    out_base = wid * B_PER_W

    def chunk_body(c, _):
        pltpu.sync_copy(
            idx_hbm.at[pl.ds(idx_row_base + c * ROWS_PER_CHUNK, ROWS_PER_CHUNK)],
            idx_v,
        )
        descs = []
        for r in range(ROWS_PER_CHUNK):
            descs.append(
                pltpu.async_copy(
                    table_hbm.at[idx_v.at[r]],
                    rows_v.at[pl.ds(r * IDX_ROW, IDX_ROW)],
                    gsem,
                )
            )
        for d in descs:
            d.wait()
        pltpu.sync_copy(rows_v, out_hbm.at[pl.ds(out_base + c * CHUNK, CHUNK)])
        return 0

    lax.fori_loop(0, N_CHUNKS, chunk_body, 0)


def _emb_lookup(idx2d, table):
    mesh = plsc.VectorSubcoreMesh(core_axis_name="c", subcore_axis_name="s")
    k = functools.partial(
        pl.kernel,
        mesh=mesh,
        out_type=jax.ShapeDtypeStruct((B, D_EMB), jnp.float32),
        scratch_types=[
            pltpu.VMEM((ROWS_PER_CHUNK, IDX_ROW), jnp.int32),
            pltpu.VMEM((CHUNK, D_EMB), jnp.float32),
            pltpu.SemaphoreType.DMA,
        ],
        compiler_params=pltpu.CompilerParams(use_tc_tiling_on_sc=False),
    )(_emb_body)
    return k(idx2d, table)


def kernel(batch_words, emb_weight):
    idx2d = batch_words.reshape(B // IDX_ROW, IDX_ROW).astype(jnp.int32)
    out = _emb_lookup(idx2d, emb_weight)
    return out.reshape(BATCH, SEQ, D_EMB)
